# baseline (speedup 1.0000x reference)
import functools

import jax
import jax.numpy as jnp
from jax import lax
from jax.experimental import pallas as pl
from jax.experimental.pallas import tpu as pltpu
from jax.experimental.pallas import tpu_sc as plsc

N_NODES = 10000
N_EDGES = 160000
HIDDEN = 32
NI = 16
D = HIDDEN * NI
QW = 128
NQ = D // QW
NC = 2
NS = 16
RPT = 640
N_PAD = NS * RPT
EPT = N_EDGES // NS
EB = 80
NB = EPT // EB
ZR = 32


def _ylm_body(ev_ref, yt_ref):
    v = ev_ref[...]
    x = v[0:1, :]
    y = v[1:2, :]
    z = v[2:3, :]
    r = jnp.sqrt(x * x + y * y + z * z)
    r = jnp.maximum(r, 1e-8)
    inv = 1.0 / r
    x = x * inv
    y = y * inv
    z = z * inv
    x2, y2, z2 = x * x, y * y, z * z
    rows = [
        0.28209479177387814 * jnp.ones_like(x),
        0.4886025119029199 * y,
        0.4886025119029199 * z,
        0.4886025119029199 * x,
        1.0925484305920792 * x * y,
        1.0925484305920792 * y * z,
        0.31539156525252005 * (3.0 * z2 - 1.0),
        1.0925484305920792 * x * z,
        0.5462742152960396 * (x2 - y2),
        0.5900435899266435 * y * (3.0 * x2 - y2),
        2.890611442640554 * x * y * z,
        0.4570457994644658 * y * (5.0 * z2 - 1.0),
        0.3731763325901154 * z * (5.0 * z2 - 3.0),
        0.4570457994644658 * x * (5.0 * z2 - 1.0),
        1.445305721320277 * z * (x2 - y2),
        0.5900435899266435 * x * (x2 - 3.0 * y2),
    ]
    yt_ref[...] = jnp.concatenate(rows, axis=0).T


def _ylm_call(evt):
    e = evt.shape[1]
    bl = 3200
    return pl.pallas_call(
        _ylm_body,
        grid=(e // bl,),
        in_specs=[pl.BlockSpec((3, bl), lambda i: (0, i))],
        out_specs=pl.BlockSpec((bl, 16), lambda i: (i, 0)),
        out_shape=jax.ShapeDtypeStruct((e, 16), jnp.float32),
    )(evt)


def _tf_body(nf_ref, w_ref, out_ref):
    t = jnp.dot(nf_ref[...].astype(jnp.bfloat16), w_ref[...],
                preferred_element_type=jnp.float32)
    for q in range(NQ):
        out_ref[q, :, :] = t[:, q * QW:(q + 1) * QW]


def _tf_call(nf_flat, w2):
    bn = 400
    return pl.pallas_call(
        _tf_body,
        grid=(N_NODES // bn,),
        in_specs=[
            pl.BlockSpec((bn, D), lambda i: (i, 0)),
            pl.BlockSpec((D, D), lambda i: (0, 0)),
        ],
        out_specs=pl.BlockSpec((NQ, bn, QW), lambda i: (0, i, 0)),
        out_shape=jax.ShapeDtypeStruct((NQ, N_PAD, QW), jnp.float32),
    )(nf_flat, w2)


def _lin_body(acc_ref, w_ref, b_ref, out_ref):
    dn = (((1,), (1,)), ((), ()))
    r = lax.dot_general(acc_ref[0].astype(jnp.bfloat16),
                        w_ref[:, pl.ds(0, QW)].astype(jnp.bfloat16), dn,
                        preferred_element_type=jnp.float32)
    for q in range(1, NQ):
        r = r + lax.dot_general(acc_ref[q].astype(jnp.bfloat16),
                                w_ref[:, pl.ds(q * QW, QW)].astype(jnp.bfloat16),
                                dn, preferred_element_type=jnp.float32)
    out_ref[...] = r + b_ref[...]


def _lin_call(acc4, w, b2):
    bn = 400
    return pl.pallas_call(
        _lin_body,
        grid=(N_NODES // bn,),
        in_specs=[
            pl.BlockSpec((NQ, bn, QW), lambda i: (0, i, 0)),
            pl.BlockSpec((D, D), lambda i: (0, 0)),
            pl.BlockSpec((1, D), lambda i: (0, 0)),
        ],
        out_specs=pl.BlockSpec((bn, D), lambda i: (i, 0)),
        out_shape=jax.ShapeDtypeStruct((N_NODES, D), jnp.float32),
    )(acc4, w, b2)


def _edge_body(table_hbm, y_hbm, ei_hbm, out_hbm,
               zbuf, sidx_all, didx0, didx1, didx2, rows0, rows1, rows2,
               yb0, yb1, yb2, acc,
               sem_g0, sem_g1, sem_g2, sem_y0, sem_y1, sem_y2,
               sem_s0, sem_s1, sem_s2):
    c = lax.axis_index("c")
    s = lax.axis_index("s")
    rows = (rows0, rows1, rows2)
    ybs = (yb0, yb1, yb2)
    didx = (didx0, didx1, didx2)
    sem_g = (sem_g0, sem_g1, sem_g2)
    sem_y = (sem_y0, sem_y1, sem_y2)
    sem_s = (sem_s0, sem_s1, sem_s2)

    def zz(i, carry):
        for j in range(QW // 16):
            zbuf[i, pl.ds(j * 16, 16)] = jnp.zeros((16,), jnp.float32)
        return carry

    lax.fori_loop(0, ZR, zz, 0)

    for jq in range(NQ // NC):
        q = c * (NQ // NC) + jq
        row0 = q * N_PAD
        pltpu.sync_copy(ei_hbm.at[pl.ds(s * EPT, EPT)], sidx_all)

        def shift(i, carry):
            sidx_all[pl.ds(i * 16, 16)] = sidx_all[pl.ds(i * 16, 16)] + row0
            return carry

        lax.fori_loop(0, EPT // 16, shift, 0)

        for k in range(RPT // ZR):
            pltpu.sync_copy(zbuf, acc.at[pl.ds(s * RPT + k * ZR, ZR)])
        plsc.subcore_barrier()

        def start_all(b, p):
            base = s * EPT + b * EB
            pltpu.async_copy(table_hbm.at[sidx_all.at[pl.ds(b * EB, EB)]],
                             rows[p], sem_g[p])
            pltpu.async_copy(y_hbm.at[pl.ds(base * NI, EB * NI)], ybs[p],
                             sem_y[p])
            pltpu.async_copy(ei_hbm.at[pl.ds(N_EDGES + base, EB)], didx[p],
                             sem_y[p])

        def wait_rows(b, p):
            pltpu.make_async_copy(
                table_hbm.at[sidx_all.at[pl.ds(b * EB, EB)]],
                rows[p], sem_g[p]).wait()

        def wait_aux(b, p):
            base = s * EPT + b * EB
            pltpu.make_async_copy(
                y_hbm.at[pl.ds(base * NI, EB * NI)], ybs[p], sem_y[p]).wait()
            pltpu.make_async_copy(
                ei_hbm.at[pl.ds(N_EDGES + base, EB)], didx[p],
                sem_y[p]).wait()

        def start_scatter(b, p):
            pltpu.async_copy(rows[p], acc.at[didx[p]], sem_s[p], add=True)

        def wait_scatter(b, p):
            pltpu.make_async_copy(
                rows[p], acc.at[didx[p]], sem_s[p]).wait()

        def scale(b, p):
            rp = rows[p]
            yp = ybs[p]

            @plsc.parallel_loop(0, EB, step=1, unroll=4)
            def edge_scale(e):
                yv = yp[pl.ds(e * NI, 16)]
                for cc in range(QW // 16):
                    rp[e, pl.ds(cc * 16, 16)] = rp[e, pl.ds(cc * 16, 16)] * yv

        start_all(0, 0)
        for b0 in (0, 1):
            start_all(b0 + 1, (b0 + 1) % 3)
            wait_rows(b0, b0 % 3)
            wait_aux(b0, b0 % 3)
            scale(b0, b0 % 3)
            start_scatter(b0, b0 % 3)

        def triple(i, carry):
            for k in range(3):
                b = 3 * i + 2 + k
                p = (2 + k) % 3
                g1 = (p + 1) % 3
                wait_scatter(b - 2, g1)
                start_all(jnp.minimum(b + 1, NB - 1), g1)
                wait_rows(b, p)
                wait_aux(b, p)
                scale(b, p)
                start_scatter(b, p)
            return carry

        lax.fori_loop(0, (NB - 2) // 3, triple, 0)
        wait_scatter(NB - 2, (NB - 2) % 3)
        wait_scatter(NB - 1, (NB - 1) % 3)
        wait_rows(NB - 1, NB % 3)
        wait_aux(NB - 1, NB % 3)
        plsc.subcore_barrier()
        pltpu.sync_copy(acc.at[pl.ds(s * RPT, RPT)],
                        out_hbm.at[pl.ds(row0 + s * RPT, RPT)])


@functools.cache
def _make_edge_call():
    return pl.kernel(
        _edge_body,
        out_type=jax.ShapeDtypeStruct((NQ * N_PAD, QW), jnp.float32),
        mesh=plsc.VectorSubcoreMesh(core_axis_name="c", subcore_axis_name="s",
                                    num_cores=NC, num_subcores=NS),
        scratch_types=[
            pltpu.VMEM((ZR, QW), jnp.float32),
            pltpu.VMEM((EPT,), jnp.int32),
            pltpu.VMEM((EB,), jnp.int32),
            pltpu.VMEM((EB,), jnp.int32),
            pltpu.VMEM((EB,), jnp.int32),
            pltpu.VMEM((EB, QW), jnp.float32),
            pltpu.VMEM((EB, QW), jnp.float32),
            pltpu.VMEM((EB, QW), jnp.float32),
            pltpu.VMEM((EB * NI,), jnp.float32),
            pltpu.VMEM((EB * NI,), jnp.float32),
            pltpu.VMEM((EB * NI,), jnp.float32),
            pltpu.VMEM_SHARED((N_PAD, QW), jnp.float32),
        ] + [pltpu.SemaphoreType.DMA] * 9,
    )


def kernel(node_features, edge_index, edge_vectors, tp_weights, lin_w, lin_b):
    n = node_features.shape[0]
    ei = edge_index.astype(jnp.int32)

    y = _ylm_call(edge_vectors.T)

    tp_t = jnp.transpose(tp_weights, (2, 1, 0))
    eye = jnp.eye(NI, dtype=jnp.float32)
    w2 = (tp_t[:, None, :, :] * eye[None, :, None, :]).reshape(D, D)

    nf_flat = node_features.reshape(n, D)
    table = _tf_call(nf_flat, w2.astype(jnp.bfloat16)).reshape(
        NQ * N_PAD, QW)

    acc = _make_edge_call()(table, y.reshape(-1), ei.reshape(-1))
    acc4 = acc.reshape(NQ, N_PAD, QW)

    out = _lin_call(acc4, lin_w, lin_b.reshape(1, D))
    return out.reshape(n, HIDDEN, NI)


# speedup vs baseline: 1.0474x; 1.0474x over previous
import functools

import jax
import jax.numpy as jnp
from jax import lax
from jax.experimental import pallas as pl
from jax.experimental.pallas import tpu as pltpu
from jax.experimental.pallas import tpu_sc as plsc

N_NODES = 10000
N_EDGES = 160000
HIDDEN = 32
NI = 16
D = HIDDEN * NI
QW = 128
NQ = D // QW
NC = 2
NS = 16
RPT = 640
N_PAD = NS * RPT
EPT = N_EDGES // NS
EB = 80
NB = EPT // EB
ZR = 8


def _ylm_body(ev_ref, yt_ref):
    v = ev_ref[...]
    x = v[0:1, :]
    y = v[1:2, :]
    z = v[2:3, :]
    r = jnp.sqrt(x * x + y * y + z * z)
    r = jnp.maximum(r, 1e-8)
    inv = 1.0 / r
    x = x * inv
    y = y * inv
    z = z * inv
    x2, y2, z2 = x * x, y * y, z * z
    rows = [
        0.28209479177387814 * jnp.ones_like(x),
        0.4886025119029199 * y,
        0.4886025119029199 * z,
        0.4886025119029199 * x,
        1.0925484305920792 * x * y,
        1.0925484305920792 * y * z,
        0.31539156525252005 * (3.0 * z2 - 1.0),
        1.0925484305920792 * x * z,
        0.5462742152960396 * (x2 - y2),
        0.5900435899266435 * y * (3.0 * x2 - y2),
        2.890611442640554 * x * y * z,
        0.4570457994644658 * y * (5.0 * z2 - 1.0),
        0.3731763325901154 * z * (5.0 * z2 - 3.0),
        0.4570457994644658 * x * (5.0 * z2 - 1.0),
        1.445305721320277 * z * (x2 - y2),
        0.5900435899266435 * x * (x2 - 3.0 * y2),
    ]
    yt_ref[...] = jnp.concatenate(rows, axis=0).T


def _ylm_call(evt):
    e = evt.shape[1]
    bl = 3200
    return pl.pallas_call(
        _ylm_body,
        grid=(e // bl,),
        in_specs=[pl.BlockSpec((3, bl), lambda i: (0, i))],
        out_specs=pl.BlockSpec((bl, 16), lambda i: (i, 0)),
        out_shape=jax.ShapeDtypeStruct((e, 16), jnp.float32),
    )(evt)


def _tf_body(nf_ref, w_ref, out_ref):
    t = jnp.dot(nf_ref[...].astype(jnp.bfloat16), w_ref[...],
                preferred_element_type=jnp.float32)
    for q in range(NQ):
        out_ref[q, :, :] = t[:, q * QW:(q + 1) * QW]


def _tf_call(nf_flat, w2):
    bn = 400
    return pl.pallas_call(
        _tf_body,
        grid=(N_NODES // bn,),
        in_specs=[
            pl.BlockSpec((bn, D), lambda i: (i, 0)),
            pl.BlockSpec((D, D), lambda i: (0, 0)),
        ],
        out_specs=pl.BlockSpec((NQ, bn, QW), lambda i: (0, i, 0)),
        out_shape=jax.ShapeDtypeStruct((NQ, N_PAD, QW), jnp.float32),
    )(nf_flat, w2)


def _lin_body(acc_ref, w_ref, b_ref, out_ref):
    dn = (((1,), (1,)), ((), ()))
    r = lax.dot_general(acc_ref[0].astype(jnp.bfloat16),
                        w_ref[:, pl.ds(0, QW)].astype(jnp.bfloat16), dn,
                        preferred_element_type=jnp.float32)
    for q in range(1, NQ):
        r = r + lax.dot_general(acc_ref[q].astype(jnp.bfloat16),
                                w_ref[:, pl.ds(q * QW, QW)].astype(jnp.bfloat16),
                                dn, preferred_element_type=jnp.float32)
    out_ref[...] = r + b_ref[...]


def _lin_call(acc4, w, b2):
    bn = 400
    return pl.pallas_call(
        _lin_body,
        grid=(N_NODES // bn,),
        in_specs=[
            pl.BlockSpec((NQ, bn, QW), lambda i: (0, i, 0)),
            pl.BlockSpec((D, D), lambda i: (0, 0)),
            pl.BlockSpec((1, D), lambda i: (0, 0)),
        ],
        out_specs=pl.BlockSpec((bn, D), lambda i: (i, 0)),
        out_shape=jax.ShapeDtypeStruct((N_NODES, D), jnp.float32),
    )(acc4, w, b2)


def _edge_body(table_hbm, y_hbm, ei_hbm, out_hbm,
               zbuf, sidx0, sidx1, sidx2, sidx3, didx0, didx1, didx2, didx3,
               rows0, rows1, rows2, rows3, yb0, yb1, yb2, yb3, acc,
               sem_g0, sem_g1, sem_g2, sem_g3, sem_y0, sem_y1, sem_y2, sem_y3,
               sem_s0, sem_s1, sem_s2, sem_s3, sem_i0, sem_i1, sem_i2, sem_i3,
               sem_z):
    c = lax.axis_index("c")
    s = lax.axis_index("s")
    rows = (rows0, rows1, rows2, rows3)
    ybs = (yb0, yb1, yb2, yb3)
    didx = (didx0, didx1, didx2, didx3)
    sidx = (sidx0, sidx1, sidx2, sidx3)
    sem_g = (sem_g0, sem_g1, sem_g2, sem_g3)
    sem_y = (sem_y0, sem_y1, sem_y2, sem_y3)
    sem_s = (sem_s0, sem_s1, sem_s2, sem_s3)
    sem_i = (sem_i0, sem_i1, sem_i2, sem_i3)

    def zz(i, carry):
        for j in range(QW // 16):
            zbuf[i, pl.ds(j * 16, 16)] = jnp.zeros((16,), jnp.float32)
        return carry

    lax.fori_loop(0, ZR, zz, 0)

    for jq in range(NQ // NC):
        q = c * (NQ // NC) + jq
        row0 = q * N_PAD

        def zs(i, carry):
            pltpu.async_copy(zbuf, acc.at[pl.ds(s * RPT + i * ZR, ZR)], sem_z)
            return carry

        lax.fori_loop(0, RPT // ZR, zs, 0)

        def zw(i, carry):
            pltpu.make_async_copy(
                zbuf, acc.at[pl.ds(s * RPT + i * ZR, ZR)], sem_z).wait()
            return carry

        lax.fori_loop(0, RPT // ZR, zw, 0)
        plsc.subcore_barrier()

        def start_sidx(b, p):
            base = s * EPT + b * EB
            pltpu.async_copy(ei_hbm.at[pl.ds(base, EB)], sidx[p], sem_i[p])

        def wait_sidx_shift(b, p):
            base = s * EPT + b * EB
            pltpu.make_async_copy(
                ei_hbm.at[pl.ds(base, EB)], sidx[p], sem_i[p]).wait()
            for j in range(EB // 16):
                sidx[p][pl.ds(j * 16, 16)] = sidx[p][pl.ds(j * 16, 16)] + row0

        def start_fetch(b, p):
            base = s * EPT + b * EB
            pltpu.async_copy(table_hbm.at[sidx[p]], rows[p], sem_g[p])
            pltpu.async_copy(y_hbm.at[pl.ds(base * NI, EB * NI)], ybs[p],
                             sem_y[p])
            pltpu.async_copy(ei_hbm.at[pl.ds(N_EDGES + base, EB)], didx[p],
                             sem_y[p])

        def wait_rows(b, p):
            pltpu.make_async_copy(table_hbm.at[sidx[p]], rows[p],
                                  sem_g[p]).wait()

        def wait_aux(b, p):
            base = s * EPT + b * EB
            pltpu.make_async_copy(
                y_hbm.at[pl.ds(base * NI, EB * NI)], ybs[p], sem_y[p]).wait()
            pltpu.make_async_copy(
                ei_hbm.at[pl.ds(N_EDGES + base, EB)], didx[p],
                sem_y[p]).wait()

        def start_scatter(b, p):
            pltpu.async_copy(rows[p], acc.at[didx[p]], sem_s[p], add=True)

        def wait_scatter(b, p):
            pltpu.make_async_copy(
                rows[p], acc.at[didx[p]], sem_s[p]).wait()

        def scale(b, p):
            rp = rows[p]
            yp = ybs[p]

            @plsc.parallel_loop(0, EB, step=1, unroll=4)
            def edge_scale(e):
                yv = yp[pl.ds(e * NI, 16)]
                for cc in range(QW // 16):
                    rp[e, pl.ds(cc * 16, 16)] = rp[e, pl.ds(cc * 16, 16)] * yv

        def phase(b, p, first):
            g2 = (p + 2) % 4
            bc = jnp.minimum(b + 2, NB - 1)
            if not first:
                wait_scatter(b - 2, g2)
            wait_sidx_shift(bc, g2)
            start_fetch(bc, g2)
            wait_rows(b, p)
            start_sidx(jnp.minimum(b + 4, NB - 1), p)
            wait_aux(b, p)
            scale(b, p)
            start_scatter(b, p)

        for b0 in range(4):
            start_sidx(b0, b0)
        for b0 in range(2):
            wait_sidx_shift(b0, b0)
            start_fetch(b0, b0)
        phase(0, 0, True)
        phase(1, 1, True)

        def quad(i, carry):
            for k in range(4):
                b = 4 * i + 2 + k
                phase(b, (2 + k) % 4, False)
            return carry

        lax.fori_loop(0, (NB - 5) // 4, quad, 0)
        for b0 in range(NB - 3, NB):
            phase(b0, b0 % 4, False)
        wait_scatter(NB - 2, (NB - 2) % 4)
        wait_scatter(NB - 1, (NB - 1) % 4)
        wait_rows(NB - 1, NB % 4)
        wait_aux(NB - 1, NB % 4)
        wait_rows(NB - 1, (NB + 1) % 4)
        wait_aux(NB - 1, (NB + 1) % 4)
        pltpu.make_async_copy(
            ei_hbm.at[pl.ds(s * EPT + (NB - 1) * EB, EB)],
            sidx[(NB - 2) % 4], sem_i[(NB - 2) % 4]).wait()
        pltpu.make_async_copy(
            ei_hbm.at[pl.ds(s * EPT + (NB - 1) * EB, EB)],
            sidx[(NB - 1) % 4], sem_i[(NB - 1) % 4]).wait()
        plsc.subcore_barrier()
        pltpu.sync_copy(acc.at[pl.ds(s * RPT, RPT)],
                        out_hbm.at[pl.ds(row0 + s * RPT, RPT)])


@functools.cache
def _make_edge_call():
    return pl.kernel(
        _edge_body,
        out_type=jax.ShapeDtypeStruct((NQ * N_PAD, QW), jnp.float32),
        mesh=plsc.VectorSubcoreMesh(core_axis_name="c", subcore_axis_name="s",
                                    num_cores=NC, num_subcores=NS),
        scratch_types=(
            [pltpu.VMEM((ZR, QW), jnp.float32)]
            + [pltpu.VMEM((EB,), jnp.int32)] * 8
            + [pltpu.VMEM((EB, QW), jnp.float32)] * 4
            + [pltpu.VMEM((EB * NI,), jnp.float32)] * 4
            + [pltpu.VMEM_SHARED((N_PAD, QW), jnp.float32)]
            + [pltpu.SemaphoreType.DMA] * 17
        ),
    )


def kernel(node_features, edge_index, edge_vectors, tp_weights, lin_w, lin_b):
    n = node_features.shape[0]
    ei = edge_index.astype(jnp.int32)

    y = _ylm_call(edge_vectors.T)

    tp_t = jnp.transpose(tp_weights, (2, 1, 0))
    eye = jnp.eye(NI, dtype=jnp.float32)
    w2 = (tp_t[:, None, :, :] * eye[None, :, None, :]).reshape(D, D)

    nf_flat = node_features.reshape(n, D)
    table = _tf_call(nf_flat, w2.astype(jnp.bfloat16)).reshape(
        NQ * N_PAD, QW)

    acc = _make_edge_call()(table, y.reshape(-1), ei.reshape(-1))
    acc4 = acc.reshape(NQ, N_PAD, QW)

    out = _lin_call(acc4, lin_w, lin_b.reshape(1, D))
    return out.reshape(n, HIDDEN, NI)
